# baseline (speedup 1.0000x reference)
import functools
import math

import jax
import jax.numpy as jnp
from jax import lax
from jax.experimental import pallas as pl
from jax.experimental.pallas import tpu as pltpu
from jax.experimental.pallas import tpu_sc as plsc

N_NODES = 50000
H = 64

NC = 2
NS = 16
NW = NC * NS

NPAD = 50176
RPT = NPAD // NS
EPAD = 802816
C = 64
SC_CH = 2048
NSCAN = EPAD // SC_CH

NB = 64
BMUL, BSH = 671, 19
_LO = [-(-v * (1 << BSH) // BMUL) for v in range(65)]
SZMIN = 781
SZMAX = 782
ACC_R = SZMAX + 1

R = 512
GRID = NPAD // R

_mesh = plsc.VectorSubcoreMesh(core_axis_name="c", subcore_axis_name="s")
_sc_params = pltpu.CompilerParams(use_tc_tiling_on_sc=False,
                                  needs_layout_passes=False)


def _vmax16(ref):
  return ref[pl.ds(0, 16)][0]


def _make_route_kernel():
  out_type = [
      jax.ShapeDtypeStruct((NB, EPAD), jnp.int32),
      jax.ShapeDtypeStruct((NB, EPAD), jnp.int32),
      jax.ShapeDtypeStruct((NB, 16), jnp.int32),
  ]
  scratch = [
      pltpu.VMEM((SC_CH,), jnp.int32),
      pltpu.VMEM((SC_CH,), jnp.int32),
      pltpu.VMEM((288,), jnp.int32),
      pltpu.VMEM((288,), jnp.int32),
      pltpu.VMEM((288,), jnp.int32),
      pltpu.VMEM((288,), jnp.int32),
      pltpu.VMEM((16,), jnp.int32),
      pltpu.VMEM((16,), jnp.int32),
  ]

  def body(dst_hbm, src_hbm, lo_tab, rd_out, rs_out, cnt_out,
           dv, sv, bd0, bs0, bd1, bs1, lv, cb):
    c = lax.axis_index("c")
    s = lax.axis_index("s")
    w = c * NS + s
    bds = (bd0, bd1)
    bss = (bs0, bs1)

    def scan_body(k, carry):
      pltpu.sync_copy(dst_hbm.at[pl.ds(k * SC_CH, SC_CH)], dv)
      pltpu.sync_copy(src_hbm.at[pl.ds(k * SC_CH, SC_CH)], sv)

      def vec_body(j, carry2):
        p0, o0, p1, o1 = carry2
        d = dv[pl.ds(j * 16, 16)]
        sc = sv[pl.ds(j * 16, 16)]
        b = (d * BMUL) >> BSH
        new = []
        for h, pos2, off2 in ((0, p0, o0), (1, p1, o1)):
          v = 2 * w + h
          m = b == v
          cnt = plsc.all_reduce_population_count(m)[0]

          @pl.when(cnt > 0)
          def _(h=h, m=m, pos2=pos2):
            key = jnp.where(m, 0, 1).astype(jnp.int32)
            _, dc, scc = lax.sort((key, d, sc), dimension=0, num_keys=1)
            bds[h][pl.ds(pos2, 16)] = dc
            bss[h][pl.ds(pos2, 16)] = scc

          pos3 = pos2 + cnt

          def flush(h=h, v=v, off2=off2):
            o = pl.multiple_of(off2, 128)
            pltpu.sync_copy(bds[h].at[pl.ds(0, 128)],
                            rd_out.at[v, pl.ds(o, 128)])
            pltpu.sync_copy(bss[h].at[pl.ds(0, 128)],
                            rs_out.at[v, pl.ds(o, 128)])
            bds[h][pl.ds(0, 16)] = bds[h][pl.ds(128, 16)]
            bss[h][pl.ds(0, 16)] = bss[h][pl.ds(128, 16)]

          pl.when(pos3 >= 128)(flush)
          rolled = pos3 >= 128
          new.append(jnp.where(rolled, pos3 - 128, pos3))
          new.append(jnp.where(rolled, off2 + 128, off2))
        return tuple(new)

      return lax.fori_loop(0, SC_CH // 16, vec_body, carry, unroll=2)

    z = jnp.int32(0)
    p0, o0, p1, o1 = lax.fori_loop(0, NSCAN, scan_body, (z, z, z, z))

    pltpu.sync_copy(lo_tab.at[2 * w], lv)
    fd0 = jnp.full((16,), 0, jnp.int32) + (_vmax16(lv) + SZMAX)
    pltpu.sync_copy(lo_tab.at[2 * w + 1], lv)
    fd1 = jnp.full((16,), 0, jnp.int32) + (_vmax16(lv) + SZMAX)
    fs = jnp.full((16,), N_NODES, jnp.int32)
    for h, pos, off, fd in ((0, p0, o0, fd0), (1, p1, o1, fd1)):
      v = 2 * w + h
      for t in range(8):
        bds[h][pl.ds(pos + t * 16, 16)] = fd
        bss[h][pl.ds(pos + t * 16, 16)] = fs
      offm = pl.multiple_of(off, 128)
      pltpu.sync_copy(bds[h].at[pl.ds(0, 128)], rd_out.at[v, pl.ds(offm, 128)])
      pltpu.sync_copy(bss[h].at[pl.ds(0, 128)], rs_out.at[v, pl.ds(offm, 128)])
      cb[pl.ds(0, 16)] = jnp.full((16,), 0, jnp.int32) + (off + 128)
      pltpu.sync_copy(cb, cnt_out.at[v])

  return pl.kernel(body, mesh=_mesh, out_type=out_type,
                   scratch_types=scratch, compiler_params=_sc_params)


def _make_edge_kernel():
  out_type = [jax.ShapeDtypeStruct((NPAD, H), jnp.float32)]
  scratch = [
      pltpu.VMEM((C,), jnp.int32), pltpu.VMEM((C,), jnp.int32),
      pltpu.VMEM((C,), jnp.int32), pltpu.VMEM((C,), jnp.int32),
      pltpu.VMEM((C, H), jnp.float32), pltpu.VMEM((C, H), jnp.float32),
      pltpu.VMEM((16,), jnp.int32),
      pltpu.VMEM((16,), jnp.int32),
      pltpu.VMEM((16,), jnp.int32),
      pltpu.VMEM((ACC_R, H), jnp.float32),
      pltpu.VMEM((ACC_R, H), jnp.float32),
      pltpu.SemaphoreType.DMA,
      pltpu.SemaphoreType.DMA,
      pltpu.SemaphoreType.DMA,
      pltpu.SemaphoreType.DMA,
  ]

  def body(rd_hbm, rs_hbm, cnt_hbm, lo_tab, a_tab, b_tab, zacc,
           s_out,
           dv0, dv1, sv0, sv1, rb0, rb1, lv, hv, cv, a_loc, acc,
           semg0, semg1, semi0, semi1):
    c = lax.axis_index("c")
    s = lax.axis_index("s")
    w = c * NS + s

    dvs = (dv0, dv1)
    svs = (sv0, sv1)
    rbs = (rb0, rb1)
    semg = (semg0, semg1)
    semi = (semi0, semi1)

    for h in range(2):
      v = 2 * w + h
      pltpu.sync_copy(lo_tab.at[v], lv)
      pltpu.sync_copy(lo_tab.at[v + 1], hv)
      pltpu.sync_copy(cnt_hbm.at[v], cv)
      lo_v = _vmax16(lv)
      sz_v = _vmax16(hv) - lo_v
      nch = _vmax16(cv) // C
      pltpu.sync_copy(a_tab.at[pl.ds(lo_v, ACC_R)], a_loc)
      pltpu.sync_copy(zacc, acc)

      def issue_idx(g, b, v=v):
        o = pl.multiple_of(g * C, C)
        pltpu.async_copy(rd_hbm.at[v, pl.ds(o, C)], dvs[b], semi[b])
        pltpu.async_copy(rs_hbm.at[v, pl.ds(o, C)], svs[b], semi[b])

      def wait_idx(b, v=v):
        o = pl.multiple_of(0, C)
        pltpu.make_async_copy(rd_hbm.at[v, pl.ds(o, C)], dvs[b],
                              semi[b]).wait()
        pltpu.make_async_copy(rs_hbm.at[v, pl.ds(o, C)], svs[b],
                              semi[b]).wait()

      def issue_gather(b):
        pltpu.async_copy(b_tab.at[svs[b]], rbs[b], semg[b])

      def wait_gather(b):
        pltpu.make_async_copy(b_tab.at[svs[b]], rbs[b], semg[b]).wait()

      issue_idx(0, 0)
      issue_idx(1, 1)
      wait_idx(0)
      issue_gather(0)

      def pair_body(k, carry, lo_v=lo_v, nch=nch,
                    issue_idx=issue_idx, wait_idx=wait_idx,
                    issue_gather=issue_gather, wait_gather=wait_gather):
        for b in range(2):
          g = 2 * k + b
          wait_gather(b)
          dv, rb = dvs[b], rbs[b]

          def group_body(q, carry2, dv=dv, rb=rb):
            dvv = dv[pl.ds(q * 16, 16)] - lo_v
            for pair in range(8):
              tvals = []
              for lane2 in range(2):
                lane = pair * 2 + lane2
                lr = dvv[lane]
                r = q * 16 + lane
                for seg in range(H // 16):
                  sl = pl.ds(seg * 16, 16)
                  tvals.append(
                      (lr, sl,
                       jnp.maximum(a_loc[lr, sl] + rb[r, sl], 0.0)))
              for lr, sl, t in tvals:
                plsc.addupdate(acc.at[lr, sl], t)
            return carry2

          lax.fori_loop(0, C // 16, group_body, 0)

          @pl.when(g + 2 < nch)
          def _(g=g, b=b):
            issue_idx(g + 2, b)

          @pl.when(g + 1 < nch)
          def _(g=g, b=b):
            wait_idx(1 - b)
            issue_gather(1 - b)
        return carry

      lax.fori_loop(0, nch // 2, pair_body, 0)

      pltpu.sync_copy(acc.at[pl.ds(0, SZMIN)], s_out.at[pl.ds(lo_v, SZMIN)])

      @pl.when(sz_v == SZMAX)
      def _(lo_v=lo_v):
        pltpu.sync_copy(acc.at[pl.ds(SZMIN, 1)],
                        s_out.at[pl.ds(lo_v + SZMIN, 1)])

  return pl.kernel(body, mesh=_mesh, out_type=out_type,
                   scratch_types=scratch, compiler_params=_sc_params)


EPT2 = EPAD // NW
NCHUNK2 = EPT2 // 128


def _make_deg_kernel():
  out_type = [
      jax.ShapeDtypeStruct((NPAD, 8), jnp.float32),
      jax.ShapeDtypeStruct((NPAD, 8), jnp.float32),
  ]
  scratch = [
      pltpu.VMEM((128,), jnp.int32),
      pltpu.VMEM((128, 8), jnp.float32),
      pltpu.VMEM_SHARED((NPAD, 8), jnp.float32),
  ]

  def body(dst_hbm, zeros_d, ones_d, d0_out, d1_out, dstv, onesv, deg_sh):
    c = lax.axis_index("c")
    s = lax.axis_index("s")
    row0 = s * RPT
    pltpu.sync_copy(zeros_d, deg_sh.at[pl.ds(row0, RPT)])
    pltpu.sync_copy(ones_d, onesv)
    plsc.subcore_barrier()

    def chunk_body(k, carry):
      off = (c * NS + s) * EPT2 + k * 128
      pltpu.sync_copy(dst_hbm.at[pl.ds(off, 128)], dstv)
      pltpu.sync_copy(onesv, deg_sh.at[dstv], add=True)
      return carry

    lax.fori_loop(0, NCHUNK2, chunk_body, 0)
    plsc.subcore_barrier()

    @pl.when(c == 0)
    def _():
      pltpu.sync_copy(deg_sh.at[pl.ds(row0, RPT)], d0_out.at[pl.ds(row0, RPT)])

    @pl.when(c == 1)
    def _():
      pltpu.sync_copy(deg_sh.at[pl.ds(row0, RPT)], d1_out.at[pl.ds(row0, RPT)])

  return pl.kernel(body, mesh=_mesh, out_type=out_type,
                   scratch_types=scratch, compiler_params=_sc_params)


_route_k = _make_route_kernel()
_edge_k = _make_edge_kernel()
_deg_k = _make_deg_kernel()


def _dot(a, b):
  return jnp.dot(a, b, preferred_element_type=jnp.float32)


def _blk(shape):
  return pl.BlockSpec(shape, lambda i: (0,) * len(shape))


def _rowblk(cols):
  return pl.BlockSpec((R, cols), lambda i: (i, 0))


def _enc_body(x_ref, w1, b1, w2, b2, wi, bi, h_ref, ie_ref):
  x = x_ref[...]
  t = jnp.maximum(_dot(x, w1[...]) + b1[...], 0.0)
  t = jnp.maximum(_dot(t, w2[...]) + b2[...], 0.0)
  ie = _dot(x, wi[...]) + bi[...]
  h_ref[...] = t + ie
  ie_ref[...] = ie


def _encoder(xp, p):
  return pl.pallas_call(
      _enc_body,
      grid=(GRID,),
      in_specs=[_rowblk(2), _blk((2, 32)), _blk((1, 32)), _blk((32, H)),
                _blk((1, H)), _blk((2, H)), _blk((1, H))],
      out_specs=[_rowblk(H), _rowblk(H)],
      out_shape=[jax.ShapeDtypeStruct((NPAD, H), jnp.float32)] * 2,
  )(xp, p['enc_w1'], p['enc_b1'].reshape(1, -1), p['enc_w2'],
    p['enc_b2'].reshape(1, -1), p['inp_w'], p['inp_b'].reshape(1, -1))


def _make_pre_body(has_cat):
  def body(h_ref, ie_ref, cw1a, cw1b, cb1, cw2, cb2, mw1a, mw1b, mb1,
           hm_ref, a_ref, b_ref):
    h = h_ref[...]
    if has_cat:
      cc = jnp.maximum(_dot(h, cw1a[...]) + _dot(ie_ref[...], cw1b[...])
                       + cb1[...], 0.0)
      cc = _dot(cc, cw2[...]) + cb2[...]
      h = jnp.maximum(cc, 0.0)
    hm_ref[...] = h
    a_ref[...] = _dot(h, mw1a[...]) + mb1[...]
    b_ref[...] = _dot(h, mw1b[...])
  return body


def _pre(h, ie, p, i, has_cat):
  return pl.pallas_call(
      _make_pre_body(has_cat),
      grid=(GRID,),
      in_specs=[_rowblk(H), _rowblk(H),
                _blk((H, H)), _blk((H, H)), _blk((1, H)),
                _blk((H, H)), _blk((1, H)),
                _blk((H, H)), _blk((H, H)), _blk((1, H))],
      out_specs=[_rowblk(H), _rowblk(H), _rowblk(H)],
      out_shape=[jax.ShapeDtypeStruct((NPAD, H), jnp.float32)] * 3,
  )(h, ie, p['cat_w1'][:H], p['cat_w1'][H:], p['cat_b1'].reshape(1, -1),
    p['cat_w2'], p['cat_b2'].reshape(1, -1),
    p['msg_w1_%d' % i][:H], p['msg_w1_%d' % i][H:],
    p['msg_b1_%d' % i].reshape(1, -1))


def _post_body(hm_ref, s_ref, dg0_ref, dg1_ref, w2, mb2,
               uw1a, uw1b, ub1, uw2, ub2, out_ref):
  hm = hm_ref[...]
  dg = dg0_ref[...][:, 0:1] + dg1_ref[...][:, 0:1]
  agg = _dot(s_ref[...], w2[...]) + dg * mb2[...]
  u = jnp.maximum(_dot(hm, uw1a[...]) + _dot(agg, uw1b[...]) + ub1[...], 0.0)
  u = _dot(u, uw2[...]) + ub2[...]
  out_ref[...] = jnp.maximum(u + hm, 0.0)


def _post(hm, sagg, dg0, dg1, p, i):
  return pl.pallas_call(
      _post_body,
      grid=(GRID,),
      in_specs=[_rowblk(H), _rowblk(H), _rowblk(8), _rowblk(8),
                _blk((H, H)), _blk((1, H)),
                _blk((H, H)), _blk((H, H)), _blk((1, H)),
                _blk((H, H)), _blk((1, H))],
      out_specs=_rowblk(H),
      out_shape=jax.ShapeDtypeStruct((NPAD, H), jnp.float32),
  )(hm, sagg, dg0, dg1,
    p['msg_w2_%d' % i], p['msg_b2_%d' % i].reshape(1, -1),
    p['upd_w1_%d' % i][:H], p['upd_w1_%d' % i][H:],
    p['upd_b1_%d' % i].reshape(1, -1),
    p['upd_w2_%d' % i], p['upd_b2_%d' % i].reshape(1, -1))


def _dec_body(h_ref, w1, b1, w2, b2, o_ref):
  d = jnp.maximum(_dot(h_ref[...], w1[...]) + b1[...], 0.0)
  z = _dot(d, w2[...]) + b2[...]
  m = jnp.max(z, axis=1, keepdims=True)
  e = jnp.exp(z - m)
  o_ref[...] = (z - m) - jnp.log(jnp.sum(e, axis=1, keepdims=True))


def _decoder(h, p):
  return pl.pallas_call(
      _dec_body,
      grid=(GRID,),
      in_specs=[_rowblk(H), _blk((H, 64)), _blk((1, 64)), _blk((64, 2)),
                _blk((1, 2))],
      out_specs=_rowblk(2),
      out_shape=jax.ShapeDtypeStruct((NPAD, 2), jnp.float32),
  )(h, p['dec_w1'], p['dec_b1'].reshape(1, -1), p['dec_w2'],
    p['dec_b2'].reshape(1, -1))


def kernel(x, edge_index, num_nodes, params):
  n = x.shape[0]
  e = edge_index.shape[1]
  p = params

  xp = jnp.pad(x, ((0, NPAD - n), (0, 0)))
  pad_e = EPAD - e
  fill = jnp.full((pad_e,), n, jnp.int32)
  src_p = jnp.concatenate([edge_index[0], fill])
  dst_p = jnp.concatenate([edge_index[1], fill])

  lo_tab = jnp.broadcast_to(
      jnp.asarray(_LO + [0] * 7, jnp.int32)[:, None], (72, 16))
  zeros_d = jnp.zeros((RPT, 8), jnp.float32)
  ones_d = jnp.ones((128, 8), jnp.float32)
  zacc = jnp.zeros((ACC_R, H), jnp.float32)

  rdst, rsrc, cnts = _route_k(dst_p, src_p, lo_tab)
  dg0, dg1 = _deg_k(dst_p, zeros_d, ones_d)
  h, ie = _encoder(xp, p)

  eff = min(8, max(4, int(math.log2(n))))
  for i in range(eff):
    hm, a_tab, b_tab = _pre(h, ie, p, i, has_cat=(i > 0))
    (sagg,) = _edge_k(rdst, rsrc, cnts, lo_tab, a_tab, b_tab, zacc)
    h = _post(hm, sagg, dg0, dg1, p, i)

  out = _decoder(h, p)
  return out[:n]


# speedup vs baseline: 1.0912x; 1.0912x over previous
import functools
import math

import jax
import jax.numpy as jnp
from jax import lax
from jax.experimental import pallas as pl
from jax.experimental.pallas import tpu as pltpu
from jax.experimental.pallas import tpu_sc as plsc

N_NODES = 50000
H = 64

NC = 2
NS = 16
NW = NC * NS

NPAD = 50176
RPT = NPAD // NS
EPAD = 802816
C = 64
SC_CH = 2048
NSCAN = EPAD // SC_CH

NB = 64
BMUL, BSH = 671, 19
_LO = [-(-v * (1 << BSH) // BMUL) for v in range(65)]
SZMIN = 781
SZMAX = 782
ACC_R = SZMAX + 1

R = 512
GRID = NPAD // R

_mesh = plsc.VectorSubcoreMesh(core_axis_name="c", subcore_axis_name="s")
_sc_params = pltpu.CompilerParams(use_tc_tiling_on_sc=False,
                                  needs_layout_passes=False)


def _vmax16(ref):
  return ref[pl.ds(0, 16)][0]


def _make_route_kernel():
  out_type = [
      jax.ShapeDtypeStruct((NB, EPAD), jnp.int32),
      jax.ShapeDtypeStruct((NB, EPAD), jnp.int32),
      jax.ShapeDtypeStruct((NB, 16), jnp.int32),
  ]
  scratch = [
      pltpu.VMEM((SC_CH,), jnp.int32),
      pltpu.VMEM((SC_CH,), jnp.int32),
      pltpu.VMEM((288,), jnp.int32),
      pltpu.VMEM((288,), jnp.int32),
      pltpu.VMEM((288,), jnp.int32),
      pltpu.VMEM((288,), jnp.int32),
      pltpu.VMEM((16,), jnp.int32),
      pltpu.VMEM((16,), jnp.int32),
  ]

  def body(dst_hbm, src_hbm, lo_tab, rd_out, rs_out, cnt_out,
           dv, sv, bd0, bs0, bd1, bs1, lv, cb):
    c = lax.axis_index("c")
    s = lax.axis_index("s")
    w = c * NS + s
    bds = (bd0, bd1)
    bss = (bs0, bs1)

    def scan_body(k, carry):
      pltpu.sync_copy(dst_hbm.at[pl.ds(k * SC_CH, SC_CH)], dv)
      pltpu.sync_copy(src_hbm.at[pl.ds(k * SC_CH, SC_CH)], sv)

      def vec_body(j, carry2):
        p0, o0, p1, o1 = carry2
        d = dv[pl.ds(j * 16, 16)]
        sc = sv[pl.ds(j * 16, 16)]
        b = (d * BMUL) >> BSH
        new = []
        for h, pos2, off2 in ((0, p0, o0), (1, p1, o1)):
          v = 2 * w + h
          m = b == v
          key = jnp.where(m, 0, 1).astype(jnp.int32)
          _, dc, scc = lax.sort((key, d, sc), dimension=0, num_keys=1)
          bds[h][pl.ds(pos2, 16)] = dc
          bss[h][pl.ds(pos2, 16)] = scc
          cnt = plsc.all_reduce_population_count(m)[0]
          pos3 = pos2 + cnt

          def flush(h=h, v=v, off2=off2):
            o = pl.multiple_of(off2, 128)
            pltpu.sync_copy(bds[h].at[pl.ds(0, 128)],
                            rd_out.at[v, pl.ds(o, 128)])
            pltpu.sync_copy(bss[h].at[pl.ds(0, 128)],
                            rs_out.at[v, pl.ds(o, 128)])
            bds[h][pl.ds(0, 16)] = bds[h][pl.ds(128, 16)]
            bss[h][pl.ds(0, 16)] = bss[h][pl.ds(128, 16)]

          pl.when(pos3 >= 128)(flush)
          rolled = pos3 >= 128
          new.append(jnp.where(rolled, pos3 - 128, pos3))
          new.append(jnp.where(rolled, off2 + 128, off2))
        return tuple(new)

      return lax.fori_loop(0, SC_CH // 16, vec_body, carry, unroll=2)

    z = jnp.int32(0)
    p0, o0, p1, o1 = lax.fori_loop(0, NSCAN, scan_body, (z, z, z, z))

    pltpu.sync_copy(lo_tab.at[2 * w], lv)
    fd0 = jnp.full((16,), 0, jnp.int32) + (_vmax16(lv) + SZMAX)
    pltpu.sync_copy(lo_tab.at[2 * w + 1], lv)
    fd1 = jnp.full((16,), 0, jnp.int32) + (_vmax16(lv) + SZMAX)
    fs = jnp.full((16,), N_NODES, jnp.int32)
    for h, pos, off, fd in ((0, p0, o0, fd0), (1, p1, o1, fd1)):
      v = 2 * w + h
      for t in range(8):
        bds[h][pl.ds(pos + t * 16, 16)] = fd
        bss[h][pl.ds(pos + t * 16, 16)] = fs
      offm = pl.multiple_of(off, 128)
      pltpu.sync_copy(bds[h].at[pl.ds(0, 128)], rd_out.at[v, pl.ds(offm, 128)])
      pltpu.sync_copy(bss[h].at[pl.ds(0, 128)], rs_out.at[v, pl.ds(offm, 128)])
      cb[pl.ds(0, 16)] = jnp.full((16,), 0, jnp.int32) + (off + 128)
      pltpu.sync_copy(cb, cnt_out.at[v])

  return pl.kernel(body, mesh=_mesh, out_type=out_type,
                   scratch_types=scratch, compiler_params=_sc_params)


def _make_edge_kernel():
  out_type = [jax.ShapeDtypeStruct((NPAD, H), jnp.float32)]
  scratch = [
      pltpu.VMEM((C,), jnp.int32), pltpu.VMEM((C,), jnp.int32),
      pltpu.VMEM((C,), jnp.int32), pltpu.VMEM((C,), jnp.int32),
      pltpu.VMEM((C, H), jnp.float32), pltpu.VMEM((C, H), jnp.float32),
      pltpu.VMEM((16,), jnp.int32),
      pltpu.VMEM((16,), jnp.int32),
      pltpu.VMEM((16,), jnp.int32),
      pltpu.VMEM((ACC_R, H), jnp.float32),
      pltpu.VMEM((ACC_R, H), jnp.float32),
      pltpu.SemaphoreType.DMA,
      pltpu.SemaphoreType.DMA,
      pltpu.SemaphoreType.DMA,
      pltpu.SemaphoreType.DMA,
  ]

  def body(rd_hbm, rs_hbm, cnt_hbm, lo_tab, a_tab, b_tab, zacc,
           s_out,
           dv0, dv1, sv0, sv1, rb0, rb1, lv, hv, cv, a_loc, acc,
           semg0, semg1, semi0, semi1):
    c = lax.axis_index("c")
    s = lax.axis_index("s")
    w = c * NS + s

    dvs = (dv0, dv1)
    svs = (sv0, sv1)
    rbs = (rb0, rb1)
    semg = (semg0, semg1)
    semi = (semi0, semi1)

    for h in range(2):
      v = 2 * w + h
      pltpu.sync_copy(lo_tab.at[v], lv)
      pltpu.sync_copy(lo_tab.at[v + 1], hv)
      pltpu.sync_copy(cnt_hbm.at[v], cv)
      lo_v = _vmax16(lv)
      sz_v = _vmax16(hv) - lo_v
      nch = _vmax16(cv) // C
      pltpu.sync_copy(a_tab.at[pl.ds(lo_v, ACC_R)], a_loc)
      pltpu.sync_copy(zacc, acc)

      def issue_idx(g, b, v=v):
        o = pl.multiple_of(g * C, C)
        pltpu.async_copy(rd_hbm.at[v, pl.ds(o, C)], dvs[b], semi[b])
        pltpu.async_copy(rs_hbm.at[v, pl.ds(o, C)], svs[b], semi[b])

      def wait_idx(b, v=v):
        o = pl.multiple_of(0, C)
        pltpu.make_async_copy(rd_hbm.at[v, pl.ds(o, C)], dvs[b],
                              semi[b]).wait()
        pltpu.make_async_copy(rs_hbm.at[v, pl.ds(o, C)], svs[b],
                              semi[b]).wait()

      def issue_gather(b):
        pltpu.async_copy(b_tab.at[svs[b]], rbs[b], semg[b])

      def wait_gather(b):
        pltpu.make_async_copy(b_tab.at[svs[b]], rbs[b], semg[b]).wait()

      issue_idx(0, 0)
      issue_idx(1, 1)
      wait_idx(0)
      issue_gather(0)

      def pair_body(k, carry, lo_v=lo_v, nch=nch,
                    issue_idx=issue_idx, wait_idx=wait_idx,
                    issue_gather=issue_gather, wait_gather=wait_gather):
        for b in range(2):
          g = 2 * k + b
          wait_gather(b)
          dv, rb = dvs[b], rbs[b]

          def group_body(q, carry2, dv=dv, rb=rb):
            dvv = dv[pl.ds(q * 16, 16)] - lo_v
            for pair in range(8):
              tvals = []
              for lane2 in range(2):
                lane = pair * 2 + lane2
                lr = dvv[lane]
                r = q * 16 + lane
                for seg in range(H // 16):
                  sl = pl.ds(seg * 16, 16)
                  tvals.append(
                      (lr, sl,
                       jnp.maximum(a_loc[lr, sl] + rb[r, sl], 0.0)))
              for lr, sl, t in tvals:
                plsc.addupdate(acc.at[lr, sl], t)
            return carry2

          lax.fori_loop(0, C // 16, group_body, 0)

          @pl.when(g + 2 < nch)
          def _(g=g, b=b):
            issue_idx(g + 2, b)

          @pl.when(g + 1 < nch)
          def _(g=g, b=b):
            wait_idx(1 - b)
            issue_gather(1 - b)
        return carry

      lax.fori_loop(0, nch // 2, pair_body, 0)

      pltpu.sync_copy(acc.at[pl.ds(0, SZMIN)], s_out.at[pl.ds(lo_v, SZMIN)])

      @pl.when(sz_v == SZMAX)
      def _(lo_v=lo_v):
        pltpu.sync_copy(acc.at[pl.ds(SZMIN, 1)],
                        s_out.at[pl.ds(lo_v + SZMIN, 1)])

  return pl.kernel(body, mesh=_mesh, out_type=out_type,
                   scratch_types=scratch, compiler_params=_sc_params)


EPT2 = EPAD // NW
NCHUNK2 = EPT2 // 128


def _make_deg_kernel():
  out_type = [
      jax.ShapeDtypeStruct((NPAD, 8), jnp.float32),
      jax.ShapeDtypeStruct((NPAD, 8), jnp.float32),
  ]
  scratch = [
      pltpu.VMEM((128,), jnp.int32),
      pltpu.VMEM((128, 8), jnp.float32),
      pltpu.VMEM_SHARED((NPAD, 8), jnp.float32),
  ]

  def body(dst_hbm, zeros_d, ones_d, d0_out, d1_out, dstv, onesv, deg_sh):
    c = lax.axis_index("c")
    s = lax.axis_index("s")
    row0 = s * RPT
    pltpu.sync_copy(zeros_d, deg_sh.at[pl.ds(row0, RPT)])
    pltpu.sync_copy(ones_d, onesv)
    plsc.subcore_barrier()

    def chunk_body(k, carry):
      off = (c * NS + s) * EPT2 + k * 128
      pltpu.sync_copy(dst_hbm.at[pl.ds(off, 128)], dstv)
      pltpu.sync_copy(onesv, deg_sh.at[dstv], add=True)
      return carry

    lax.fori_loop(0, NCHUNK2, chunk_body, 0)
    plsc.subcore_barrier()

    @pl.when(c == 0)
    def _():
      pltpu.sync_copy(deg_sh.at[pl.ds(row0, RPT)], d0_out.at[pl.ds(row0, RPT)])

    @pl.when(c == 1)
    def _():
      pltpu.sync_copy(deg_sh.at[pl.ds(row0, RPT)], d1_out.at[pl.ds(row0, RPT)])

  return pl.kernel(body, mesh=_mesh, out_type=out_type,
                   scratch_types=scratch, compiler_params=_sc_params)


_route_k = _make_route_kernel()
_edge_k = _make_edge_kernel()
_deg_k = _make_deg_kernel()


def _dot(a, b):
  return jnp.dot(a, b, preferred_element_type=jnp.float32)


def _blk(shape):
  return pl.BlockSpec(shape, lambda i: (0,) * len(shape))


def _rowblk(cols):
  return pl.BlockSpec((R, cols), lambda i: (i, 0))


def _enc_body(x_ref, w1, b1, w2, b2, wi, bi, h_ref, ie_ref):
  x = x_ref[...]
  t = jnp.maximum(_dot(x, w1[...]) + b1[...], 0.0)
  t = jnp.maximum(_dot(t, w2[...]) + b2[...], 0.0)
  ie = _dot(x, wi[...]) + bi[...]
  h_ref[...] = t + ie
  ie_ref[...] = ie


def _encoder(xp, p):
  return pl.pallas_call(
      _enc_body,
      grid=(GRID,),
      in_specs=[_rowblk(2), _blk((2, 32)), _blk((1, 32)), _blk((32, H)),
                _blk((1, H)), _blk((2, H)), _blk((1, H))],
      out_specs=[_rowblk(H), _rowblk(H)],
      out_shape=[jax.ShapeDtypeStruct((NPAD, H), jnp.float32)] * 2,
  )(xp, p['enc_w1'], p['enc_b1'].reshape(1, -1), p['enc_w2'],
    p['enc_b2'].reshape(1, -1), p['inp_w'], p['inp_b'].reshape(1, -1))


def _make_pre_body(has_cat):
  def body(h_ref, ie_ref, cw1a, cw1b, cb1, cw2, cb2, mw1a, mw1b, mb1,
           hm_ref, a_ref, b_ref):
    h = h_ref[...]
    if has_cat:
      cc = jnp.maximum(_dot(h, cw1a[...]) + _dot(ie_ref[...], cw1b[...])
                       + cb1[...], 0.0)
      cc = _dot(cc, cw2[...]) + cb2[...]
      h = jnp.maximum(cc, 0.0)
    hm_ref[...] = h
    a_ref[...] = _dot(h, mw1a[...]) + mb1[...]
    b_ref[...] = _dot(h, mw1b[...])
  return body


def _pre(h, ie, p, i, has_cat):
  return pl.pallas_call(
      _make_pre_body(has_cat),
      grid=(GRID,),
      in_specs=[_rowblk(H), _rowblk(H),
                _blk((H, H)), _blk((H, H)), _blk((1, H)),
                _blk((H, H)), _blk((1, H)),
                _blk((H, H)), _blk((H, H)), _blk((1, H))],
      out_specs=[_rowblk(H), _rowblk(H), _rowblk(H)],
      out_shape=[jax.ShapeDtypeStruct((NPAD, H), jnp.float32)] * 3,
  )(h, ie, p['cat_w1'][:H], p['cat_w1'][H:], p['cat_b1'].reshape(1, -1),
    p['cat_w2'], p['cat_b2'].reshape(1, -1),
    p['msg_w1_%d' % i][:H], p['msg_w1_%d' % i][H:],
    p['msg_b1_%d' % i].reshape(1, -1))


def _post_body(hm_ref, s_ref, dg0_ref, dg1_ref, w2, mb2,
               uw1a, uw1b, ub1, uw2, ub2, out_ref):
  hm = hm_ref[...]
  dg = dg0_ref[...][:, 0:1] + dg1_ref[...][:, 0:1]
  agg = _dot(s_ref[...], w2[...]) + dg * mb2[...]
  u = jnp.maximum(_dot(hm, uw1a[...]) + _dot(agg, uw1b[...]) + ub1[...], 0.0)
  u = _dot(u, uw2[...]) + ub2[...]
  out_ref[...] = jnp.maximum(u + hm, 0.0)


def _post(hm, sagg, dg0, dg1, p, i):
  return pl.pallas_call(
      _post_body,
      grid=(GRID,),
      in_specs=[_rowblk(H), _rowblk(H), _rowblk(8), _rowblk(8),
                _blk((H, H)), _blk((1, H)),
                _blk((H, H)), _blk((H, H)), _blk((1, H)),
                _blk((H, H)), _blk((1, H))],
      out_specs=_rowblk(H),
      out_shape=jax.ShapeDtypeStruct((NPAD, H), jnp.float32),
  )(hm, sagg, dg0, dg1,
    p['msg_w2_%d' % i], p['msg_b2_%d' % i].reshape(1, -1),
    p['upd_w1_%d' % i][:H], p['upd_w1_%d' % i][H:],
    p['upd_b1_%d' % i].reshape(1, -1),
    p['upd_w2_%d' % i], p['upd_b2_%d' % i].reshape(1, -1))


def _postpre_body(hm_ref, s_ref, dg0_ref, dg1_ref, ie_ref,
                  w2, mb2, uw1a, uw1b, ub1, uw2, ub2,
                  cw1a, cw1b, cb1, cw2, cb2, mw1a, mw1b, mb1,
                  h_ref, a_ref, b_ref):
  hm = hm_ref[...]
  dg = dg0_ref[...][:, 0:1] + dg1_ref[...][:, 0:1]
  agg = _dot(s_ref[...], w2[...]) + dg * mb2[...]
  u = jnp.maximum(_dot(hm, uw1a[...]) + _dot(agg, uw1b[...]) + ub1[...], 0.0)
  u = _dot(u, uw2[...]) + ub2[...]
  hh = jnp.maximum(u + hm, 0.0)
  cc = jnp.maximum(_dot(hh, cw1a[...]) + _dot(ie_ref[...], cw1b[...])
                   + cb1[...], 0.0)
  cc = _dot(cc, cw2[...]) + cb2[...]
  hmn = jnp.maximum(cc, 0.0)
  h_ref[...] = hmn
  a_ref[...] = _dot(hmn, mw1a[...]) + mb1[...]
  b_ref[...] = _dot(hmn, mw1b[...])


def _postpre(hm, sagg, dg0, dg1, ie, p, i):
  j = i + 1
  return pl.pallas_call(
      _postpre_body,
      grid=(GRID,),
      in_specs=[_rowblk(H), _rowblk(H), _rowblk(8), _rowblk(8), _rowblk(H),
                _blk((H, H)), _blk((1, H)),
                _blk((H, H)), _blk((H, H)), _blk((1, H)),
                _blk((H, H)), _blk((1, H)),
                _blk((H, H)), _blk((H, H)), _blk((1, H)),
                _blk((H, H)), _blk((1, H)),
                _blk((H, H)), _blk((H, H)), _blk((1, H))],
      out_specs=[_rowblk(H), _rowblk(H), _rowblk(H)],
      out_shape=[jax.ShapeDtypeStruct((NPAD, H), jnp.float32)] * 3,
  )(hm, sagg, dg0, dg1, ie,
    p['msg_w2_%d' % i], p['msg_b2_%d' % i].reshape(1, -1),
    p['upd_w1_%d' % i][:H], p['upd_w1_%d' % i][H:],
    p['upd_b1_%d' % i].reshape(1, -1),
    p['upd_w2_%d' % i], p['upd_b2_%d' % i].reshape(1, -1),
    p['cat_w1'][:H], p['cat_w1'][H:], p['cat_b1'].reshape(1, -1),
    p['cat_w2'], p['cat_b2'].reshape(1, -1),
    p['msg_w1_%d' % j][:H], p['msg_w1_%d' % j][H:],
    p['msg_b1_%d' % j].reshape(1, -1))


def _dec_body(h_ref, w1, b1, w2, b2, o_ref):
  d = jnp.maximum(_dot(h_ref[...], w1[...]) + b1[...], 0.0)
  z = _dot(d, w2[...]) + b2[...]
  m = jnp.max(z, axis=1, keepdims=True)
  e = jnp.exp(z - m)
  o_ref[...] = (z - m) - jnp.log(jnp.sum(e, axis=1, keepdims=True))


def _decoder(h, p):
  return pl.pallas_call(
      _dec_body,
      grid=(GRID,),
      in_specs=[_rowblk(H), _blk((H, 64)), _blk((1, 64)), _blk((64, 2)),
                _blk((1, 2))],
      out_specs=_rowblk(2),
      out_shape=jax.ShapeDtypeStruct((NPAD, 2), jnp.float32),
  )(h, p['dec_w1'], p['dec_b1'].reshape(1, -1), p['dec_w2'],
    p['dec_b2'].reshape(1, -1))


def kernel(x, edge_index, num_nodes, params):
  n = x.shape[0]
  e = edge_index.shape[1]
  p = params

  xp = jnp.pad(x, ((0, NPAD - n), (0, 0)))
  pad_e = EPAD - e
  fill = jnp.full((pad_e,), n, jnp.int32)
  src_p = jnp.concatenate([edge_index[0], fill])
  dst_p = jnp.concatenate([edge_index[1], fill])

  lo_tab = jnp.broadcast_to(
      jnp.asarray(_LO + [0] * 7, jnp.int32)[:, None], (72, 16))
  zeros_d = jnp.zeros((RPT, 8), jnp.float32)
  ones_d = jnp.ones((128, 8), jnp.float32)
  zacc = jnp.zeros((ACC_R, H), jnp.float32)

  rdst, rsrc, cnts = _route_k(dst_p, src_p, lo_tab)
  dg0, dg1 = _deg_k(dst_p, zeros_d, ones_d)
  h, ie = _encoder(xp, p)

  eff = min(8, max(4, int(math.log2(n))))
  hm, a_tab, b_tab = _pre(h, ie, p, 0, has_cat=False)
  for i in range(eff):
    (sagg,) = _edge_k(rdst, rsrc, cnts, lo_tab, a_tab, b_tab, zacc)
    if i < eff - 1:
      hm, a_tab, b_tab = _postpre(hm, sagg, dg0, dg1, ie, p, i)
    else:
      h = _post(hm, sagg, dg0, dg1, p, i)

  out = _decoder(h, p)
  return out[:n]


# speedup vs baseline: 1.0914x; 1.0002x over previous
import functools
import math

import jax
import jax.numpy as jnp
from jax import lax
from jax.experimental import pallas as pl
from jax.experimental.pallas import tpu as pltpu
from jax.experimental.pallas import tpu_sc as plsc

N_NODES = 50000
H = 64

NC = 2
NS = 16
NW = NC * NS

NPAD = 50176
RPT = NPAD // NS
EPAD = 802816
C = 64
SC_CH = 2048
NSCAN = EPAD // SC_CH

NB = 64
BMUL, BSH = 671, 19
_LO = [-(-v * (1 << BSH) // BMUL) for v in range(65)]
SZMIN = 781
SZMAX = 782
ACC_R = SZMAX + 1

R = 512
GRID = NPAD // R

_mesh = plsc.VectorSubcoreMesh(core_axis_name="c", subcore_axis_name="s")
_sc_params = pltpu.CompilerParams(use_tc_tiling_on_sc=False,
                                  needs_layout_passes=False)


def _vmax16(ref):
  return ref[pl.ds(0, 16)][0]


def _make_route_kernel():
  out_type = [
      jax.ShapeDtypeStruct((NB, EPAD), jnp.int32),
      jax.ShapeDtypeStruct((NB, EPAD), jnp.int32),
      jax.ShapeDtypeStruct((NB, 16), jnp.int32),
  ]
  scratch = [
      pltpu.VMEM((SC_CH,), jnp.int32),
      pltpu.VMEM((SC_CH,), jnp.int32),
      pltpu.VMEM((288,), jnp.int32),
      pltpu.VMEM((288,), jnp.int32),
      pltpu.VMEM((288,), jnp.int32),
      pltpu.VMEM((288,), jnp.int32),
      pltpu.VMEM((16,), jnp.int32),
      pltpu.VMEM((16,), jnp.int32),
  ]

  def body(dst_hbm, src_hbm, lo_tab, rd_out, rs_out, cnt_out,
           dv, sv, bd0, bs0, bd1, bs1, lv, cb):
    c = lax.axis_index("c")
    s = lax.axis_index("s")
    w = c * NS + s
    bds = (bd0, bd1)
    bss = (bs0, bs1)

    def scan_body(k, carry):
      pltpu.sync_copy(dst_hbm.at[pl.ds(k * SC_CH, SC_CH)], dv)
      pltpu.sync_copy(src_hbm.at[pl.ds(k * SC_CH, SC_CH)], sv)

      def vec_body(j, carry2):
        p0, o0, p1, o1 = carry2
        d = dv[pl.ds(j * 16, 16)]
        sc = sv[pl.ds(j * 16, 16)]
        b = (d * BMUL) >> BSH
        new = []
        for h, pos2, off2 in ((0, p0, o0), (1, p1, o1)):
          v = 2 * w + h
          m = b == v
          key = jnp.where(m, 0, 1).astype(jnp.int32)
          _, dc, scc = lax.sort((key, d, sc), dimension=0, num_keys=1)
          bds[h][pl.ds(pos2, 16)] = dc
          bss[h][pl.ds(pos2, 16)] = scc
          cnt = plsc.all_reduce_population_count(m)[0]
          pos3 = pos2 + cnt

          def flush(h=h, v=v, off2=off2):
            o = pl.multiple_of(off2, 128)
            pltpu.sync_copy(bds[h].at[pl.ds(0, 128)],
                            rd_out.at[v, pl.ds(o, 128)])
            pltpu.sync_copy(bss[h].at[pl.ds(0, 128)],
                            rs_out.at[v, pl.ds(o, 128)])
            bds[h][pl.ds(0, 16)] = bds[h][pl.ds(128, 16)]
            bss[h][pl.ds(0, 16)] = bss[h][pl.ds(128, 16)]

          pl.when(pos3 >= 128)(flush)
          rolled = pos3 >= 128
          new.append(jnp.where(rolled, pos3 - 128, pos3))
          new.append(jnp.where(rolled, off2 + 128, off2))
        return tuple(new)

      return lax.fori_loop(0, SC_CH // 16, vec_body, carry, unroll=4)

    z = jnp.int32(0)
    p0, o0, p1, o1 = lax.fori_loop(0, NSCAN, scan_body, (z, z, z, z))

    pltpu.sync_copy(lo_tab.at[2 * w], lv)
    fd0 = jnp.full((16,), 0, jnp.int32) + (_vmax16(lv) + SZMAX)
    pltpu.sync_copy(lo_tab.at[2 * w + 1], lv)
    fd1 = jnp.full((16,), 0, jnp.int32) + (_vmax16(lv) + SZMAX)
    fs = jnp.full((16,), N_NODES, jnp.int32)
    for h, pos, off, fd in ((0, p0, o0, fd0), (1, p1, o1, fd1)):
      v = 2 * w + h
      for t in range(8):
        bds[h][pl.ds(pos + t * 16, 16)] = fd
        bss[h][pl.ds(pos + t * 16, 16)] = fs
      offm = pl.multiple_of(off, 128)
      pltpu.sync_copy(bds[h].at[pl.ds(0, 128)], rd_out.at[v, pl.ds(offm, 128)])
      pltpu.sync_copy(bss[h].at[pl.ds(0, 128)], rs_out.at[v, pl.ds(offm, 128)])
      cb[pl.ds(0, 16)] = jnp.full((16,), 0, jnp.int32) + (off + 128)
      pltpu.sync_copy(cb, cnt_out.at[v])

  return pl.kernel(body, mesh=_mesh, out_type=out_type,
                   scratch_types=scratch, compiler_params=_sc_params)


def _make_edge_kernel():
  out_type = [jax.ShapeDtypeStruct((NPAD, H), jnp.float32)]
  scratch = [
      pltpu.VMEM((C,), jnp.int32), pltpu.VMEM((C,), jnp.int32),
      pltpu.VMEM((C,), jnp.int32), pltpu.VMEM((C,), jnp.int32),
      pltpu.VMEM((C, H), jnp.float32), pltpu.VMEM((C, H), jnp.float32),
      pltpu.VMEM((16,), jnp.int32),
      pltpu.VMEM((16,), jnp.int32),
      pltpu.VMEM((16,), jnp.int32),
      pltpu.VMEM((ACC_R, H), jnp.float32),
      pltpu.VMEM((ACC_R, H), jnp.float32),
      pltpu.SemaphoreType.DMA,
      pltpu.SemaphoreType.DMA,
      pltpu.SemaphoreType.DMA,
      pltpu.SemaphoreType.DMA,
  ]

  def body(rd_hbm, rs_hbm, cnt_hbm, lo_tab, a_tab, b_tab, zacc,
           s_out,
           dv0, dv1, sv0, sv1, rb0, rb1, lv, hv, cv, a_loc, acc,
           semg0, semg1, semi0, semi1):
    c = lax.axis_index("c")
    s = lax.axis_index("s")
    w = c * NS + s

    dvs = (dv0, dv1)
    svs = (sv0, sv1)
    rbs = (rb0, rb1)
    semg = (semg0, semg1)
    semi = (semi0, semi1)

    for h in range(2):
      v = 2 * w + h
      pltpu.sync_copy(lo_tab.at[v], lv)
      pltpu.sync_copy(lo_tab.at[v + 1], hv)
      pltpu.sync_copy(cnt_hbm.at[v], cv)
      lo_v = _vmax16(lv)
      sz_v = _vmax16(hv) - lo_v
      nch = _vmax16(cv) // C
      pltpu.sync_copy(a_tab.at[pl.ds(lo_v, ACC_R)], a_loc)
      pltpu.sync_copy(zacc, acc)

      def issue_idx(g, b, v=v):
        o = pl.multiple_of(g * C, C)
        pltpu.async_copy(rd_hbm.at[v, pl.ds(o, C)], dvs[b], semi[b])
        pltpu.async_copy(rs_hbm.at[v, pl.ds(o, C)], svs[b], semi[b])

      def wait_idx(b, v=v):
        o = pl.multiple_of(0, C)
        pltpu.make_async_copy(rd_hbm.at[v, pl.ds(o, C)], dvs[b],
                              semi[b]).wait()
        pltpu.make_async_copy(rs_hbm.at[v, pl.ds(o, C)], svs[b],
                              semi[b]).wait()

      def issue_gather(b):
        pltpu.async_copy(b_tab.at[svs[b]], rbs[b], semg[b])

      def wait_gather(b):
        pltpu.make_async_copy(b_tab.at[svs[b]], rbs[b], semg[b]).wait()

      issue_idx(0, 0)
      issue_idx(1, 1)
      wait_idx(0)
      issue_gather(0)

      def pair_body(k, carry, lo_v=lo_v, nch=nch,
                    issue_idx=issue_idx, wait_idx=wait_idx,
                    issue_gather=issue_gather, wait_gather=wait_gather):
        for b in range(2):
          g = 2 * k + b
          wait_gather(b)
          dv, rb = dvs[b], rbs[b]

          def group_body(q, carry2, dv=dv, rb=rb):
            dvv = dv[pl.ds(q * 16, 16)] - lo_v
            for pair in range(8):
              tvals = []
              for lane2 in range(2):
                lane = pair * 2 + lane2
                lr = dvv[lane]
                r = q * 16 + lane
                for seg in range(H // 16):
                  sl = pl.ds(seg * 16, 16)
                  tvals.append(
                      (lr, sl,
                       jnp.maximum(a_loc[lr, sl] + rb[r, sl], 0.0)))
              for lr, sl, t in tvals:
                plsc.addupdate(acc.at[lr, sl], t)
            return carry2

          lax.fori_loop(0, C // 16, group_body, 0)

          @pl.when(g + 2 < nch)
          def _(g=g, b=b):
            issue_idx(g + 2, b)

          @pl.when(g + 1 < nch)
          def _(g=g, b=b):
            wait_idx(1 - b)
            issue_gather(1 - b)
        return carry

      lax.fori_loop(0, nch // 2, pair_body, 0)

      pltpu.sync_copy(acc.at[pl.ds(0, SZMIN)], s_out.at[pl.ds(lo_v, SZMIN)])

      @pl.when(sz_v == SZMAX)
      def _(lo_v=lo_v):
        pltpu.sync_copy(acc.at[pl.ds(SZMIN, 1)],
                        s_out.at[pl.ds(lo_v + SZMIN, 1)])

  return pl.kernel(body, mesh=_mesh, out_type=out_type,
                   scratch_types=scratch, compiler_params=_sc_params)


EPT2 = EPAD // NW
NCHUNK2 = EPT2 // 128


def _make_deg_kernel():
  out_type = [
      jax.ShapeDtypeStruct((NPAD, 8), jnp.float32),
      jax.ShapeDtypeStruct((NPAD, 8), jnp.float32),
  ]
  scratch = [
      pltpu.VMEM((128,), jnp.int32),
      pltpu.VMEM((128, 8), jnp.float32),
      pltpu.VMEM_SHARED((NPAD, 8), jnp.float32),
  ]

  def body(dst_hbm, zeros_d, ones_d, d0_out, d1_out, dstv, onesv, deg_sh):
    c = lax.axis_index("c")
    s = lax.axis_index("s")
    row0 = s * RPT
    pltpu.sync_copy(zeros_d, deg_sh.at[pl.ds(row0, RPT)])
    pltpu.sync_copy(ones_d, onesv)
    plsc.subcore_barrier()

    def chunk_body(k, carry):
      off = (c * NS + s) * EPT2 + k * 128
      pltpu.sync_copy(dst_hbm.at[pl.ds(off, 128)], dstv)
      pltpu.sync_copy(onesv, deg_sh.at[dstv], add=True)
      return carry

    lax.fori_loop(0, NCHUNK2, chunk_body, 0)
    plsc.subcore_barrier()

    @pl.when(c == 0)
    def _():
      pltpu.sync_copy(deg_sh.at[pl.ds(row0, RPT)], d0_out.at[pl.ds(row0, RPT)])

    @pl.when(c == 1)
    def _():
      pltpu.sync_copy(deg_sh.at[pl.ds(row0, RPT)], d1_out.at[pl.ds(row0, RPT)])

  return pl.kernel(body, mesh=_mesh, out_type=out_type,
                   scratch_types=scratch, compiler_params=_sc_params)


_route_k = _make_route_kernel()
_edge_k = _make_edge_kernel()
_deg_k = _make_deg_kernel()


def _dot(a, b):
  return jnp.dot(a, b, preferred_element_type=jnp.float32)


def _blk(shape):
  return pl.BlockSpec(shape, lambda i: (0,) * len(shape))


def _rowblk(cols):
  return pl.BlockSpec((R, cols), lambda i: (i, 0))


def _enc_body(x_ref, w1, b1, w2, b2, wi, bi, h_ref, ie_ref):
  x = x_ref[...]
  t = jnp.maximum(_dot(x, w1[...]) + b1[...], 0.0)
  t = jnp.maximum(_dot(t, w2[...]) + b2[...], 0.0)
  ie = _dot(x, wi[...]) + bi[...]
  h_ref[...] = t + ie
  ie_ref[...] = ie


def _encoder(xp, p):
  return pl.pallas_call(
      _enc_body,
      grid=(GRID,),
      in_specs=[_rowblk(2), _blk((2, 32)), _blk((1, 32)), _blk((32, H)),
                _blk((1, H)), _blk((2, H)), _blk((1, H))],
      out_specs=[_rowblk(H), _rowblk(H)],
      out_shape=[jax.ShapeDtypeStruct((NPAD, H), jnp.float32)] * 2,
  )(xp, p['enc_w1'], p['enc_b1'].reshape(1, -1), p['enc_w2'],
    p['enc_b2'].reshape(1, -1), p['inp_w'], p['inp_b'].reshape(1, -1))


def _make_pre_body(has_cat):
  def body(h_ref, ie_ref, cw1a, cw1b, cb1, cw2, cb2, mw1a, mw1b, mb1,
           hm_ref, a_ref, b_ref):
    h = h_ref[...]
    if has_cat:
      cc = jnp.maximum(_dot(h, cw1a[...]) + _dot(ie_ref[...], cw1b[...])
                       + cb1[...], 0.0)
      cc = _dot(cc, cw2[...]) + cb2[...]
      h = jnp.maximum(cc, 0.0)
    hm_ref[...] = h
    a_ref[...] = _dot(h, mw1a[...]) + mb1[...]
    b_ref[...] = _dot(h, mw1b[...])
  return body


def _pre(h, ie, p, i, has_cat):
  return pl.pallas_call(
      _make_pre_body(has_cat),
      grid=(GRID,),
      in_specs=[_rowblk(H), _rowblk(H),
                _blk((H, H)), _blk((H, H)), _blk((1, H)),
                _blk((H, H)), _blk((1, H)),
                _blk((H, H)), _blk((H, H)), _blk((1, H))],
      out_specs=[_rowblk(H), _rowblk(H), _rowblk(H)],
      out_shape=[jax.ShapeDtypeStruct((NPAD, H), jnp.float32)] * 3,
  )(h, ie, p['cat_w1'][:H], p['cat_w1'][H:], p['cat_b1'].reshape(1, -1),
    p['cat_w2'], p['cat_b2'].reshape(1, -1),
    p['msg_w1_%d' % i][:H], p['msg_w1_%d' % i][H:],
    p['msg_b1_%d' % i].reshape(1, -1))


def _post_body(hm_ref, s_ref, dg0_ref, dg1_ref, w2, mb2,
               uw1a, uw1b, ub1, uw2, ub2, out_ref):
  hm = hm_ref[...]
  dg = dg0_ref[...][:, 0:1] + dg1_ref[...][:, 0:1]
  agg = _dot(s_ref[...], w2[...]) + dg * mb2[...]
  u = jnp.maximum(_dot(hm, uw1a[...]) + _dot(agg, uw1b[...]) + ub1[...], 0.0)
  u = _dot(u, uw2[...]) + ub2[...]
  out_ref[...] = jnp.maximum(u + hm, 0.0)


def _post(hm, sagg, dg0, dg1, p, i):
  return pl.pallas_call(
      _post_body,
      grid=(GRID,),
      in_specs=[_rowblk(H), _rowblk(H), _rowblk(8), _rowblk(8),
                _blk((H, H)), _blk((1, H)),
                _blk((H, H)), _blk((H, H)), _blk((1, H)),
                _blk((H, H)), _blk((1, H))],
      out_specs=_rowblk(H),
      out_shape=jax.ShapeDtypeStruct((NPAD, H), jnp.float32),
  )(hm, sagg, dg0, dg1,
    p['msg_w2_%d' % i], p['msg_b2_%d' % i].reshape(1, -1),
    p['upd_w1_%d' % i][:H], p['upd_w1_%d' % i][H:],
    p['upd_b1_%d' % i].reshape(1, -1),
    p['upd_w2_%d' % i], p['upd_b2_%d' % i].reshape(1, -1))


def _postpre_body(hm_ref, s_ref, dg0_ref, dg1_ref, ie_ref,
                  w2, mb2, uw1a, uw1b, ub1, uw2, ub2,
                  cw1a, cw1b, cb1, cw2, cb2, mw1a, mw1b, mb1,
                  h_ref, a_ref, b_ref):
  hm = hm_ref[...]
  dg = dg0_ref[...][:, 0:1] + dg1_ref[...][:, 0:1]
  agg = _dot(s_ref[...], w2[...]) + dg * mb2[...]
  u = jnp.maximum(_dot(hm, uw1a[...]) + _dot(agg, uw1b[...]) + ub1[...], 0.0)
  u = _dot(u, uw2[...]) + ub2[...]
  hh = jnp.maximum(u + hm, 0.0)
  cc = jnp.maximum(_dot(hh, cw1a[...]) + _dot(ie_ref[...], cw1b[...])
                   + cb1[...], 0.0)
  cc = _dot(cc, cw2[...]) + cb2[...]
  hmn = jnp.maximum(cc, 0.0)
  h_ref[...] = hmn
  a_ref[...] = _dot(hmn, mw1a[...]) + mb1[...]
  b_ref[...] = _dot(hmn, mw1b[...])


def _postpre(hm, sagg, dg0, dg1, ie, p, i):
  j = i + 1
  return pl.pallas_call(
      _postpre_body,
      grid=(GRID,),
      in_specs=[_rowblk(H), _rowblk(H), _rowblk(8), _rowblk(8), _rowblk(H),
                _blk((H, H)), _blk((1, H)),
                _blk((H, H)), _blk((H, H)), _blk((1, H)),
                _blk((H, H)), _blk((1, H)),
                _blk((H, H)), _blk((H, H)), _blk((1, H)),
                _blk((H, H)), _blk((1, H)),
                _blk((H, H)), _blk((H, H)), _blk((1, H))],
      out_specs=[_rowblk(H), _rowblk(H), _rowblk(H)],
      out_shape=[jax.ShapeDtypeStruct((NPAD, H), jnp.float32)] * 3,
  )(hm, sagg, dg0, dg1, ie,
    p['msg_w2_%d' % i], p['msg_b2_%d' % i].reshape(1, -1),
    p['upd_w1_%d' % i][:H], p['upd_w1_%d' % i][H:],
    p['upd_b1_%d' % i].reshape(1, -1),
    p['upd_w2_%d' % i], p['upd_b2_%d' % i].reshape(1, -1),
    p['cat_w1'][:H], p['cat_w1'][H:], p['cat_b1'].reshape(1, -1),
    p['cat_w2'], p['cat_b2'].reshape(1, -1),
    p['msg_w1_%d' % j][:H], p['msg_w1_%d' % j][H:],
    p['msg_b1_%d' % j].reshape(1, -1))


def _dec_body(h_ref, w1, b1, w2, b2, o_ref):
  d = jnp.maximum(_dot(h_ref[...], w1[...]) + b1[...], 0.0)
  z = _dot(d, w2[...]) + b2[...]
  m = jnp.max(z, axis=1, keepdims=True)
  e = jnp.exp(z - m)
  o_ref[...] = (z - m) - jnp.log(jnp.sum(e, axis=1, keepdims=True))


def _decoder(h, p):
  return pl.pallas_call(
      _dec_body,
      grid=(GRID,),
      in_specs=[_rowblk(H), _blk((H, 64)), _blk((1, 64)), _blk((64, 2)),
                _blk((1, 2))],
      out_specs=_rowblk(2),
      out_shape=jax.ShapeDtypeStruct((NPAD, 2), jnp.float32),
  )(h, p['dec_w1'], p['dec_b1'].reshape(1, -1), p['dec_w2'],
    p['dec_b2'].reshape(1, -1))


def kernel(x, edge_index, num_nodes, params):
  n = x.shape[0]
  e = edge_index.shape[1]
  p = params

  xp = jnp.pad(x, ((0, NPAD - n), (0, 0)))
  pad_e = EPAD - e
  fill = jnp.full((pad_e,), n, jnp.int32)
  src_p = jnp.concatenate([edge_index[0], fill])
  dst_p = jnp.concatenate([edge_index[1], fill])

  lo_tab = jnp.broadcast_to(
      jnp.asarray(_LO + [0] * 7, jnp.int32)[:, None], (72, 16))
  zeros_d = jnp.zeros((RPT, 8), jnp.float32)
  ones_d = jnp.ones((128, 8), jnp.float32)
  zacc = jnp.zeros((ACC_R, H), jnp.float32)

  rdst, rsrc, cnts = _route_k(dst_p, src_p, lo_tab)
  dg0, dg1 = _deg_k(dst_p, zeros_d, ones_d)
  h, ie = _encoder(xp, p)

  eff = min(8, max(4, int(math.log2(n))))
  hm, a_tab, b_tab = _pre(h, ie, p, 0, has_cat=False)
  for i in range(eff):
    (sagg,) = _edge_k(rdst, rsrc, cnts, lo_tab, a_tab, b_tab, zacc)
    if i < eff - 1:
      hm, a_tab, b_tab = _postpre(hm, sagg, dg0, dg1, ie, p, i)
    else:
      h = _post(hm, sagg, dg0, dg1, p, i)

  out = _decoder(h, p)
  return out[:n]


# speedup vs baseline: 1.0917x; 1.0003x over previous
import math

import jax
import jax.numpy as jnp
from jax import lax
from jax.experimental import pallas as pl
from jax.experimental.pallas import tpu as pltpu
from jax.experimental.pallas import tpu_sc as plsc

N_NODES = 50000
H = 64

NC = 2
NS = 16
NW = NC * NS

NPAD = 50176
RPT = NPAD // NS
EPAD = 802816
C = 64
SC_CH = 2048
NSCAN = EPAD // SC_CH

NB = 64
BMUL, BSH = 671, 19
_LO = [-(-v * (1 << BSH) // BMUL) for v in range(65)]
SZMIN = 781
SZMAX = 782
ACC_R = SZMAX + 1

R = 512
GRID = NPAD // R

_mesh = plsc.VectorSubcoreMesh(core_axis_name="c", subcore_axis_name="s")
_sc_params = pltpu.CompilerParams(use_tc_tiling_on_sc=False,
                                  needs_layout_passes=False)


def _vmax16(ref):
  return ref[pl.ds(0, 16)][0]


def _make_route_kernel():
  out_type = [
      jax.ShapeDtypeStruct((NB, EPAD), jnp.int32),
      jax.ShapeDtypeStruct((NB, EPAD), jnp.int32),
      jax.ShapeDtypeStruct((NB, 16), jnp.int32),
  ]
  scratch = [
      pltpu.VMEM((SC_CH,), jnp.int32),
      pltpu.VMEM((SC_CH,), jnp.int32),
      pltpu.VMEM((288,), jnp.int32),
      pltpu.VMEM((288,), jnp.int32),
      pltpu.VMEM((288,), jnp.int32),
      pltpu.VMEM((288,), jnp.int32),
      pltpu.VMEM((16,), jnp.int32),
      pltpu.VMEM((16,), jnp.int32),
  ]

  def body(dst_hbm, src_hbm, lo_tab, rd_out, rs_out, cnt_out,
           dv, sv, bd0, bs0, bd1, bs1, lv, cb):
    c = lax.axis_index("c")
    s = lax.axis_index("s")
    w = c * NS + s
    bds = (bd0, bd1)
    bss = (bs0, bs1)

    def scan_body(k, carry):
      pltpu.sync_copy(dst_hbm.at[pl.ds(k * SC_CH, SC_CH)], dv)
      pltpu.sync_copy(src_hbm.at[pl.ds(k * SC_CH, SC_CH)], sv)

      def vec_body(j, carry2):
        p0, o0, p1, o1 = carry2
        d = dv[pl.ds(j * 16, 16)]
        sc = sv[pl.ds(j * 16, 16)]
        b = (d * BMUL) >> BSH
        new = []
        for h, pos2, off2 in ((0, p0, o0), (1, p1, o1)):
          v = 2 * w + h
          m = b == v
          key = jnp.where(m, 0, 1).astype(jnp.int32)
          _, dc, scc = lax.sort((key, d, sc), dimension=0, num_keys=1)
          bds[h][pl.ds(pos2, 16)] = dc
          bss[h][pl.ds(pos2, 16)] = scc
          cnt = plsc.all_reduce_population_count(m)[0]
          pos3 = pos2 + cnt

          def flush(h=h, v=v, off2=off2):
            o = pl.multiple_of(off2, 128)
            pltpu.sync_copy(bds[h].at[pl.ds(0, 128)],
                            rd_out.at[v, pl.ds(o, 128)])
            pltpu.sync_copy(bss[h].at[pl.ds(0, 128)],
                            rs_out.at[v, pl.ds(o, 128)])
            bds[h][pl.ds(0, 16)] = bds[h][pl.ds(128, 16)]
            bss[h][pl.ds(0, 16)] = bss[h][pl.ds(128, 16)]

          pl.when(pos3 >= 128)(flush)
          rolled = pos3 >= 128
          new.append(jnp.where(rolled, pos3 - 128, pos3))
          new.append(jnp.where(rolled, off2 + 128, off2))
        return tuple(new)

      return lax.fori_loop(0, SC_CH // 16, vec_body, carry, unroll=4)

    z = jnp.int32(0)
    p0, o0, p1, o1 = lax.fori_loop(0, NSCAN, scan_body, (z, z, z, z))

    pltpu.sync_copy(lo_tab.at[2 * w], lv)
    fd0 = jnp.full((16,), 0, jnp.int32) + (_vmax16(lv) + SZMAX)
    pltpu.sync_copy(lo_tab.at[2 * w + 1], lv)
    fd1 = jnp.full((16,), 0, jnp.int32) + (_vmax16(lv) + SZMAX)
    fs = jnp.full((16,), N_NODES, jnp.int32)
    for h, pos, off, fd in ((0, p0, o0, fd0), (1, p1, o1, fd1)):
      v = 2 * w + h
      for t in range(8):
        bds[h][pl.ds(pos + t * 16, 16)] = fd
        bss[h][pl.ds(pos + t * 16, 16)] = fs
      offm = pl.multiple_of(off, 128)
      pltpu.sync_copy(bds[h].at[pl.ds(0, 128)], rd_out.at[v, pl.ds(offm, 128)])
      pltpu.sync_copy(bss[h].at[pl.ds(0, 128)], rs_out.at[v, pl.ds(offm, 128)])
      cb[pl.ds(0, 16)] = jnp.full((16,), 0, jnp.int32) + (off + 128)
      pltpu.sync_copy(cb, cnt_out.at[v])

  return pl.kernel(body, mesh=_mesh, out_type=out_type,
                   scratch_types=scratch, compiler_params=_sc_params)


def _make_edge_kernel():
  out_type = [jax.ShapeDtypeStruct((NPAD, H), jnp.float32)]
  scratch = [
      pltpu.VMEM((C,), jnp.int32), pltpu.VMEM((C,), jnp.int32),
      pltpu.VMEM((C,), jnp.int32), pltpu.VMEM((C,), jnp.int32),
      pltpu.VMEM((C, H), jnp.float32), pltpu.VMEM((C, H), jnp.float32),
      pltpu.VMEM((16,), jnp.int32),
      pltpu.VMEM((16,), jnp.int32),
      pltpu.VMEM((16,), jnp.int32),
      pltpu.VMEM((ACC_R, H), jnp.float32),
      pltpu.VMEM((ACC_R, H), jnp.float32),
      pltpu.SemaphoreType.DMA,
      pltpu.SemaphoreType.DMA,
      pltpu.SemaphoreType.DMA,
      pltpu.SemaphoreType.DMA,
  ]

  def body(rd_hbm, rs_hbm, cnt_hbm, lo_tab, a_tab, b_tab, zacc,
           s_out,
           dv0, dv1, sv0, sv1, rb0, rb1, lv, hv, cv, a_loc, acc,
           semg0, semg1, semi0, semi1):
    c = lax.axis_index("c")
    s = lax.axis_index("s")
    w = c * NS + s

    dvs = (dv0, dv1)
    svs = (sv0, sv1)
    rbs = (rb0, rb1)
    semg = (semg0, semg1)
    semi = (semi0, semi1)

    for h in range(2):
      v = 2 * w + h
      pltpu.sync_copy(lo_tab.at[v], lv)
      pltpu.sync_copy(lo_tab.at[v + 1], hv)
      pltpu.sync_copy(cnt_hbm.at[v], cv)
      lo_v = _vmax16(lv)
      sz_v = _vmax16(hv) - lo_v
      nch = _vmax16(cv) // C
      pltpu.sync_copy(a_tab.at[pl.ds(lo_v, ACC_R)], a_loc)
      pltpu.sync_copy(zacc, acc)

      def issue_idx(g, b, v=v):
        o = pl.multiple_of(g * C, C)
        pltpu.async_copy(rd_hbm.at[v, pl.ds(o, C)], dvs[b], semi[b])
        pltpu.async_copy(rs_hbm.at[v, pl.ds(o, C)], svs[b], semi[b])

      def wait_idx(b, v=v):
        o = pl.multiple_of(0, C)
        pltpu.make_async_copy(rd_hbm.at[v, pl.ds(o, C)], dvs[b],
                              semi[b]).wait()
        pltpu.make_async_copy(rs_hbm.at[v, pl.ds(o, C)], svs[b],
                              semi[b]).wait()

      def issue_gather(b):
        pltpu.async_copy(b_tab.at[svs[b]], rbs[b], semg[b])

      def wait_gather(b):
        pltpu.make_async_copy(b_tab.at[svs[b]], rbs[b], semg[b]).wait()

      issue_idx(0, 0)
      issue_idx(1, 1)
      wait_idx(0)
      issue_gather(0)

      def pair_body(k, carry, lo_v=lo_v, nch=nch,
                    issue_idx=issue_idx, wait_idx=wait_idx,
                    issue_gather=issue_gather, wait_gather=wait_gather):
        for b in range(2):
          g = 2 * k + b
          wait_gather(b)
          dv, rb = dvs[b], rbs[b]

          def group_body(q, carry2, dv=dv, rb=rb):
            dvv = dv[pl.ds(q * 16, 16)] - lo_v
            for pair in range(8):
              tvals = []
              for lane2 in range(2):
                lane = pair * 2 + lane2
                lr = dvv[lane]
                r = q * 16 + lane
                for seg in range(H // 16):
                  sl = pl.ds(seg * 16, 16)
                  tvals.append(
                      (lr, sl,
                       jnp.maximum(a_loc[lr, sl] + rb[r, sl], 0.0)))
              for lr, sl, t in tvals:
                plsc.addupdate(acc.at[lr, sl], t)
            return carry2

          lax.fori_loop(0, C // 16, group_body, 0)

          @pl.when(g + 2 < nch)
          def _(g=g, b=b):
            issue_idx(g + 2, b)

          @pl.when(g + 1 < nch)
          def _(g=g, b=b):
            wait_idx(1 - b)
            issue_gather(1 - b)
        return carry

      lax.fori_loop(0, nch // 2, pair_body, 0)

      pltpu.sync_copy(acc.at[pl.ds(0, SZMIN)], s_out.at[pl.ds(lo_v, SZMIN)])

      @pl.when(sz_v == SZMAX)
      def _(lo_v=lo_v):
        pltpu.sync_copy(acc.at[pl.ds(SZMIN, 1)],
                        s_out.at[pl.ds(lo_v + SZMIN, 1)])

  return pl.kernel(body, mesh=_mesh, out_type=out_type,
                   scratch_types=scratch, compiler_params=_sc_params)


EPT2 = EPAD // NW
NCHUNK2 = EPT2 // 128


def _make_deg_kernel():
  out_type = [
      jax.ShapeDtypeStruct((NPAD, 8), jnp.float32),
      jax.ShapeDtypeStruct((NPAD, 8), jnp.float32),
  ]
  scratch = [
      pltpu.VMEM((128,), jnp.int32),
      pltpu.VMEM((128, 8), jnp.float32),
      pltpu.VMEM_SHARED((NPAD, 8), jnp.float32),
  ]

  def body(dst_hbm, zeros_d, ones_d, d0_out, d1_out, dstv, onesv, deg_sh):
    c = lax.axis_index("c")
    s = lax.axis_index("s")
    row0 = s * RPT
    pltpu.sync_copy(zeros_d, deg_sh.at[pl.ds(row0, RPT)])
    pltpu.sync_copy(ones_d, onesv)
    plsc.subcore_barrier()

    def chunk_body(k, carry):
      off = (c * NS + s) * EPT2 + k * 128
      pltpu.sync_copy(dst_hbm.at[pl.ds(off, 128)], dstv)
      pltpu.sync_copy(onesv, deg_sh.at[dstv], add=True)
      return carry

    lax.fori_loop(0, NCHUNK2, chunk_body, 0)
    plsc.subcore_barrier()

    @pl.when(c == 0)
    def _():
      pltpu.sync_copy(deg_sh.at[pl.ds(row0, RPT)], d0_out.at[pl.ds(row0, RPT)])

    @pl.when(c == 1)
    def _():
      pltpu.sync_copy(deg_sh.at[pl.ds(row0, RPT)], d1_out.at[pl.ds(row0, RPT)])

  return pl.kernel(body, mesh=_mesh, out_type=out_type,
                   scratch_types=scratch, compiler_params=_sc_params)


_route_k = _make_route_kernel()
_edge_k = _make_edge_kernel()
_deg_k = _make_deg_kernel()


def _dot(a, b):
  return jnp.dot(a, b, preferred_element_type=jnp.float32)


def _blk(shape):
  return pl.BlockSpec(shape, lambda i: (0,) * len(shape))


def _rowblk(cols):
  return pl.BlockSpec((R, cols), lambda i: (i, 0))


def _enc_body(x_ref, w1, b1, w2, b2, wi, bi, h_ref, ie_ref):
  x = x_ref[...]
  t = jnp.maximum(_dot(x, w1[...]) + b1[...], 0.0)
  t = jnp.maximum(_dot(t, w2[...]) + b2[...], 0.0)
  ie = _dot(x, wi[...]) + bi[...]
  h_ref[...] = t + ie
  ie_ref[...] = ie


def _encoder(xp, p):
  return pl.pallas_call(
      _enc_body,
      grid=(GRID,),
      in_specs=[_rowblk(2), _blk((2, 32)), _blk((1, 32)), _blk((32, H)),
                _blk((1, H)), _blk((2, H)), _blk((1, H))],
      out_specs=[_rowblk(H), _rowblk(H)],
      out_shape=[jax.ShapeDtypeStruct((NPAD, H), jnp.float32)] * 2,
  )(xp, p['enc_w1'], p['enc_b1'].reshape(1, -1), p['enc_w2'],
    p['enc_b2'].reshape(1, -1), p['inp_w'], p['inp_b'].reshape(1, -1))


def _make_pre_body(has_cat):
  def body(h_ref, ie_ref, cw1a, cw1b, cb1, cw2, cb2, mw1a, mw1b, mb1,
           hm_ref, a_ref, b_ref):
    h = h_ref[...]
    if has_cat:
      cc = jnp.maximum(_dot(h, cw1a[...]) + _dot(ie_ref[...], cw1b[...])
                       + cb1[...], 0.0)
      cc = _dot(cc, cw2[...]) + cb2[...]
      h = jnp.maximum(cc, 0.0)
    hm_ref[...] = h
    a_ref[...] = _dot(h, mw1a[...]) + mb1[...]
    b_ref[...] = _dot(h, mw1b[...])
  return body


def _pre(h, ie, p, i, has_cat):
  return pl.pallas_call(
      _make_pre_body(has_cat),
      grid=(GRID,),
      in_specs=[_rowblk(H), _rowblk(H),
                _blk((H, H)), _blk((H, H)), _blk((1, H)),
                _blk((H, H)), _blk((1, H)),
                _blk((H, H)), _blk((H, H)), _blk((1, H))],
      out_specs=[_rowblk(H), _rowblk(H), _rowblk(H)],
      out_shape=[jax.ShapeDtypeStruct((NPAD, H), jnp.float32)] * 3,
  )(h, ie, p['cat_w1'][:H], p['cat_w1'][H:], p['cat_b1'].reshape(1, -1),
    p['cat_w2'], p['cat_b2'].reshape(1, -1),
    p['msg_w1_%d' % i][:H], p['msg_w1_%d' % i][H:],
    p['msg_b1_%d' % i].reshape(1, -1))


def _post_body(hm_ref, s_ref, dg0_ref, dg1_ref, w2, mb2,
               uw1a, uw1b, ub1, uw2, ub2, out_ref):
  hm = hm_ref[...]
  dg = dg0_ref[...][:, 0:1] + dg1_ref[...][:, 0:1]
  agg = _dot(s_ref[...], w2[...]) + dg * mb2[...]
  u = jnp.maximum(_dot(hm, uw1a[...]) + _dot(agg, uw1b[...]) + ub1[...], 0.0)
  u = _dot(u, uw2[...]) + ub2[...]
  out_ref[...] = jnp.maximum(u + hm, 0.0)


def _post(hm, sagg, dg0, dg1, p, i):
  return pl.pallas_call(
      _post_body,
      grid=(GRID,),
      in_specs=[_rowblk(H), _rowblk(H), _rowblk(8), _rowblk(8),
                _blk((H, H)), _blk((1, H)),
                _blk((H, H)), _blk((H, H)), _blk((1, H)),
                _blk((H, H)), _blk((1, H))],
      out_specs=_rowblk(H),
      out_shape=jax.ShapeDtypeStruct((NPAD, H), jnp.float32),
  )(hm, sagg, dg0, dg1,
    p['msg_w2_%d' % i], p['msg_b2_%d' % i].reshape(1, -1),
    p['upd_w1_%d' % i][:H], p['upd_w1_%d' % i][H:],
    p['upd_b1_%d' % i].reshape(1, -1),
    p['upd_w2_%d' % i], p['upd_b2_%d' % i].reshape(1, -1))


def _postpre_body(hm_ref, s_ref, dg0_ref, dg1_ref, ie_ref,
                  w2, mb2, uw1a, uw1b, ub1, uw2, ub2,
                  cw1a, cw1b, cb1, cw2, cb2, mw1a, mw1b, mb1,
                  h_ref, a_ref, b_ref):
  hm = hm_ref[...]
  dg = dg0_ref[...][:, 0:1] + dg1_ref[...][:, 0:1]
  agg = _dot(s_ref[...], w2[...]) + dg * mb2[...]
  u = jnp.maximum(_dot(hm, uw1a[...]) + _dot(agg, uw1b[...]) + ub1[...], 0.0)
  u = _dot(u, uw2[...]) + ub2[...]
  hh = jnp.maximum(u + hm, 0.0)
  cc = jnp.maximum(_dot(hh, cw1a[...]) + _dot(ie_ref[...], cw1b[...])
                   + cb1[...], 0.0)
  cc = _dot(cc, cw2[...]) + cb2[...]
  hmn = jnp.maximum(cc, 0.0)
  h_ref[...] = hmn
  a_ref[...] = _dot(hmn, mw1a[...]) + mb1[...]
  b_ref[...] = _dot(hmn, mw1b[...])


def _postpre(hm, sagg, dg0, dg1, ie, p, i):
  j = i + 1
  return pl.pallas_call(
      _postpre_body,
      grid=(GRID,),
      in_specs=[_rowblk(H), _rowblk(H), _rowblk(8), _rowblk(8), _rowblk(H),
                _blk((H, H)), _blk((1, H)),
                _blk((H, H)), _blk((H, H)), _blk((1, H)),
                _blk((H, H)), _blk((1, H)),
                _blk((H, H)), _blk((H, H)), _blk((1, H)),
                _blk((H, H)), _blk((1, H)),
                _blk((H, H)), _blk((H, H)), _blk((1, H))],
      out_specs=[_rowblk(H), _rowblk(H), _rowblk(H)],
      out_shape=[jax.ShapeDtypeStruct((NPAD, H), jnp.float32)] * 3,
  )(hm, sagg, dg0, dg1, ie,
    p['msg_w2_%d' % i], p['msg_b2_%d' % i].reshape(1, -1),
    p['upd_w1_%d' % i][:H], p['upd_w1_%d' % i][H:],
    p['upd_b1_%d' % i].reshape(1, -1),
    p['upd_w2_%d' % i], p['upd_b2_%d' % i].reshape(1, -1),
    p['cat_w1'][:H], p['cat_w1'][H:], p['cat_b1'].reshape(1, -1),
    p['cat_w2'], p['cat_b2'].reshape(1, -1),
    p['msg_w1_%d' % j][:H], p['msg_w1_%d' % j][H:],
    p['msg_b1_%d' % j].reshape(1, -1))


def _dec_body(h_ref, w1, b1, w2, b2, o_ref):
  d = jnp.maximum(_dot(h_ref[...], w1[...]) + b1[...], 0.0)
  z = _dot(d, w2[...]) + b2[...]
  m = jnp.max(z, axis=1, keepdims=True)
  e = jnp.exp(z - m)
  o_ref[...] = (z - m) - jnp.log(jnp.sum(e, axis=1, keepdims=True))


def _decoder(h, p):
  return pl.pallas_call(
      _dec_body,
      grid=(GRID,),
      in_specs=[_rowblk(H), _blk((H, 64)), _blk((1, 64)), _blk((64, 2)),
                _blk((1, 2))],
      out_specs=_rowblk(2),
      out_shape=jax.ShapeDtypeStruct((NPAD, 2), jnp.float32),
  )(h, p['dec_w1'], p['dec_b1'].reshape(1, -1), p['dec_w2'],
    p['dec_b2'].reshape(1, -1))


def kernel(x, edge_index, num_nodes, params):
  n = x.shape[0]
  e = edge_index.shape[1]
  p = params

  xp = jnp.pad(x, ((0, NPAD - n), (0, 0)))
  pad_e = EPAD - e
  fill = jnp.full((pad_e,), n, jnp.int32)
  src_p = jnp.concatenate([edge_index[0], fill])
  dst_p = jnp.concatenate([edge_index[1], fill])

  lo_tab = jnp.broadcast_to(
      jnp.asarray(_LO + [0] * 7, jnp.int32)[:, None], (72, 16))
  zeros_d = jnp.zeros((RPT, 8), jnp.float32)
  ones_d = jnp.ones((128, 8), jnp.float32)
  zacc = jnp.zeros((ACC_R, H), jnp.float32)

  rdst, rsrc, cnts = _route_k(dst_p, src_p, lo_tab)
  dg0, dg1 = _deg_k(dst_p, zeros_d, ones_d)
  h, ie = _encoder(xp, p)

  eff = min(8, max(4, int(math.log2(n))))
  hm, a_tab, b_tab = _pre(h, ie, p, 0, has_cat=False)
  for i in range(eff):
    (sagg,) = _edge_k(rdst, rsrc, cnts, lo_tab, a_tab, b_tab, zacc)
    if i < eff - 1:
      hm, a_tab, b_tab = _postpre(hm, sagg, dg0, dg1, ie, p, i)
    else:
      h = _post(hm, sagg, dg0, dg1, p, i)

  out = _decoder(h, p)
  return out[:n]
